# baseline (speedup 1.0000x reference)
import functools

import jax
import jax.numpy as jnp
from jax.experimental import pallas as pl
from jax.experimental.pallas import tpu as pltpu

_KS = (1, 3, 5, 7, 9, 11, 13)
_NK = len(_KS)
_PADR = 8
_PADL = 128
_NEG = -jnp.inf


def _means_body(x_ref, o_ref):
    s = jnp.sum(x_ref[...], axis=(1, 2))
    o_ref[0, 0, :] = s


def _router_body(content_ref, w1_ref, b1_ref, w2_ref, b2_ref,
                 alpha_ref, nlev_ref):
    content = content_ref[...]
    hidden = jnp.maximum(
        jax.lax.dot_general(content, w1_ref[...],
                            (((1,), (1,)), ((), ())),
                            preferred_element_type=jnp.float32)
        + b1_ref[0, :][None, :], 0.0)
    glob = jax.lax.dot_general(hidden, w2_ref[...],
                               (((1,), (1,)), ((), ())),
                               preferred_element_type=jnp.float32) \
        + b2_ref[0, :][None, :]
    e = jnp.maximum(glob, 0.0)
    q_s = jnp.clip(jnp.floor(e), 0.0, float(_NK - 2))
    w_big = e - q_s
    w_small = (q_s + 1.0) - e
    for i in range(_NK):
        fi = float(i)
        alpha = jnp.where(q_s == fi, w_small, 0.0) \
            + jnp.where(q_s == fi - 1.0, w_big, 0.0)
        alpha_ref[i, :, :] = alpha
    nlev_ref[...] = (q_s + 1.0).astype(jnp.int32)


def _pool_body(alpha_ref, nlev_ref, x_ref, o_ref, a_ref, b_ref, *, h, w, c,
               cb):
    g = pl.program_id(0)

    hp = h + 2 * _PADR
    wp = w + 2 * _PADL

    a_ref[:, 0:_PADR, :] = jnp.full((cb, _PADR, wp), _NEG, jnp.float32)
    a_ref[:, _PADR + h:, :] = jnp.full((cb, _PADR, wp), _NEG, jnp.float32)
    a_ref[:, :, _PADL - 8:_PADL] = jnp.full((cb, hp, 8), _NEG, jnp.float32)
    a_ref[:, :, _PADL + w:_PADL + w + 8] = jnp.full((cb, hp, 8), _NEG,
                                                    jnp.float32)
    x = x_ref[...]
    a_ref[:, _PADR:_PADR + h, _PADL:_PADL + w] = x

    nlevs = []
    nlev_all = 0
    for j in range(cb):
        bc = g * cb + j
        bi = bc // c
        ci = bc % c
        a0 = alpha_ref[0, bi, ci]
        o_ref[j] = (1.0 + a0) * x[j]
        nl = nlev_ref[bi, ci]
        nlevs.append((nl, bi, ci))
        nlev_all = jnp.maximum(nlev_all, nl)

    for i in range(1, _NK):
        r = (_NK - 1) - i

        @pl.when(i <= nlev_all)
        def _():
            lo = _PADR - r - 1
            hi = _PADR + h + r + 1
            rl = _PADR - r
            rh = _PADR + h + r
            cl = _PADL - r
            ch = _PADL + w + r
            v = a_ref[...]
            b_ref[:, lo:hi, cl:ch] = jnp.maximum(
                jnp.maximum(v[:, lo:hi, cl - 1:ch - 1], v[:, lo:hi, cl:ch]),
                v[:, lo:hi, cl + 1:ch + 1])
            u = b_ref[...]
            a_ref[:, rl:rh, cl:ch] = jnp.maximum(
                jnp.maximum(u[:, lo:rh - 1, cl:ch], u[:, rl:rh, cl:ch]),
                u[:, rl + 1:hi, cl:ch])

            for j in range(cb):
                nl, bi, ci = nlevs[j]

                @pl.when((i >= nl - 1) & (i <= nl))
                def _():
                    ai = alpha_ref[i, bi, ci]
                    o_ref[j] = o_ref[j] + ai * a_ref[j, _PADR:_PADR + h,
                                                     _PADL:_PADL + w]


def kernel(x, W1, b1, W2, b2):
    b, c, h, w = x.shape
    cr = W1.shape[0]
    bc = b * c
    xf = x.reshape(bc, h, w)

    cb = 16
    assert bc % cb == 0
    sums = pl.pallas_call(
        _means_body,
        grid=(bc // cb,),
        in_specs=[pl.BlockSpec((cb, h, w), lambda i: (i, 0, 0))],
        out_specs=pl.BlockSpec((1, 1, cb), lambda i: (i, 0, 0)),
        out_shape=jax.ShapeDtypeStruct((bc // cb, 1, cb), jnp.float32),
    )(xf)
    content = sums.reshape(b, c) * (1.0 / (h * w))

    alpha, nlev = pl.pallas_call(
        _router_body,
        out_shape=(jax.ShapeDtypeStruct((_NK, b, c), jnp.float32),
                   jax.ShapeDtypeStruct((b, c), jnp.int32)),
    )(content, W1, b1.reshape(1, cr), W2, b2.reshape(1, c))

    hp, wp = h + 2 * _PADR, w + 2 * _PADL
    pb = 2
    out = pl.pallas_call(
        functools.partial(_pool_body, h=h, w=w, c=c, cb=pb),
        grid=(bc // pb,),
        in_specs=[
            pl.BlockSpec(memory_space=pltpu.SMEM),
            pl.BlockSpec(memory_space=pltpu.SMEM),
            pl.BlockSpec((pb, h, w), lambda i: (i, 0, 0)),
        ],
        out_specs=pl.BlockSpec((pb, h, w), lambda i: (i, 0, 0)),
        out_shape=jax.ShapeDtypeStruct((bc, h, w), jnp.float32),
        scratch_shapes=[pltpu.VMEM((pb, hp, wp), jnp.float32),
                        pltpu.VMEM((pb, hp, wp), jnp.float32)],
    )(alpha, nlev, xf)
    return out.reshape(b, c, h, w)


# speedup vs baseline: 1.1477x; 1.1477x over previous
import functools

import jax
import jax.numpy as jnp
from jax.experimental import pallas as pl
from jax.experimental.pallas import tpu as pltpu

_KS = (1, 3, 5, 7, 9, 11, 13)
_NK = len(_KS)
_PADR = 8
_PADL = 128
_NEG = -jnp.inf


def _means_body(x_ref, o_ref):
    s = jnp.sum(x_ref[...], axis=(1, 2))
    o_ref[0, 0, :] = s


def _router_body(content_ref, w1_ref, b1_ref, w2_ref, b2_ref,
                 alpha_ref, nlev_ref):
    content = content_ref[...]
    hidden = jnp.maximum(
        jax.lax.dot_general(content, w1_ref[...],
                            (((1,), (1,)), ((), ())),
                            preferred_element_type=jnp.float32)
        + b1_ref[0, :][None, :], 0.0)
    glob = jax.lax.dot_general(hidden, w2_ref[...],
                               (((1,), (1,)), ((), ())),
                               preferred_element_type=jnp.float32) \
        + b2_ref[0, :][None, :]
    e = jnp.maximum(glob, 0.0)
    q_s = jnp.clip(jnp.floor(e), 0.0, float(_NK - 2))
    w_big = e - q_s
    w_small = (q_s + 1.0) - e
    for i in range(_NK):
        fi = float(i)
        alpha = jnp.where(q_s == fi, w_small, 0.0) \
            + jnp.where(q_s == fi - 1.0, w_big, 0.0)
        alpha_ref[i, :, :] = alpha
    nlev_ref[...] = (q_s + 1.0).astype(jnp.int32)


def _pool_body(alpha_ref, nlev_ref, x_ref, o_ref, a_ref, b_ref, *, h, w, c,
               cb):
    g = pl.program_id(0)

    hp = h + 2 * _PADR
    wp = w + 2 * _PADL

    a_ref[:, 0:_PADR, :] = jnp.full((cb, _PADR, wp), _NEG, jnp.float32)
    a_ref[:, _PADR + h:, :] = jnp.full((cb, _PADR, wp), _NEG, jnp.float32)
    a_ref[:, :, _PADL - 8:_PADL] = jnp.full((cb, hp, 8), _NEG, jnp.float32)
    a_ref[:, :, _PADL + w:_PADL + w + 8] = jnp.full((cb, hp, 8), _NEG,
                                                    jnp.float32)
    x = x_ref[...]
    a_ref[:, _PADR:_PADR + h, _PADL:_PADL + w] = x

    nlevs = []
    nlev_all = 0
    for j in range(cb):
        bc = g * cb + j
        bi = bc // c
        ci = bc % c
        a0 = alpha_ref[0, bi, ci]
        o_ref[j] = (1.0 + a0) * x[j]
        nl = nlev_ref[bi, ci]
        nlevs.append((nl, bi, ci))
        nlev_all = jnp.maximum(nlev_all, nl)

    for i in range(1, _NK):
        r = (_NK - 1) - i

        @pl.when(i <= nlev_all)
        def _():
            lo = _PADR - r - 1
            hi = _PADR + h + r + 1
            rl = _PADR - r
            rh = _PADR + h + r
            cl = _PADL - r
            ch = _PADL + w + r
            v = a_ref[...]
            b_ref[:, lo:hi, cl:ch] = jnp.maximum(
                jnp.maximum(v[:, lo:hi, cl - 1:ch - 1], v[:, lo:hi, cl:ch]),
                v[:, lo:hi, cl + 1:ch + 1])
            u = b_ref[...]
            a_ref[:, rl:rh, cl:ch] = jnp.maximum(
                jnp.maximum(u[:, lo:rh - 1, cl:ch], u[:, rl:rh, cl:ch]),
                u[:, rl + 1:hi, cl:ch])

            for j in range(cb):
                nl, bi, ci = nlevs[j]

                @pl.when((i >= nl - 1) & (i <= nl))
                def _():
                    ai = alpha_ref[i, bi, ci]
                    o_ref[j] = o_ref[j] + ai * a_ref[j, _PADR:_PADR + h,
                                                     _PADL:_PADL + w]


def kernel(x, W1, b1, W2, b2):
    b, c, h, w = x.shape
    cr = W1.shape[0]
    bc = b * c
    xf = x.reshape(bc, h, w)

    cb = 16
    assert bc % cb == 0
    sums = pl.pallas_call(
        _means_body,
        grid=(bc // cb,),
        in_specs=[pl.BlockSpec((cb, h, w), lambda i: (i, 0, 0))],
        out_specs=pl.BlockSpec((1, 1, cb), lambda i: (i, 0, 0)),
        out_shape=jax.ShapeDtypeStruct((bc // cb, 1, cb), jnp.float32),
    )(xf)
    content = sums.reshape(b, c) * (1.0 / (h * w))

    alpha, nlev = pl.pallas_call(
        _router_body,
        out_shape=(jax.ShapeDtypeStruct((_NK, b, c), jnp.float32),
                   jax.ShapeDtypeStruct((b, c), jnp.int32)),
    )(content, W1, b1.reshape(1, cr), W2, b2.reshape(1, c))

    hp, wp = h + 2 * _PADR, w + 2 * _PADL
    pb = 1
    out = pl.pallas_call(
        functools.partial(_pool_body, h=h, w=w, c=c, cb=pb),
        grid=(bc // pb,),
        in_specs=[
            pl.BlockSpec(memory_space=pltpu.SMEM),
            pl.BlockSpec(memory_space=pltpu.SMEM),
            pl.BlockSpec((pb, h, w), lambda i: (i, 0, 0)),
        ],
        out_specs=pl.BlockSpec((pb, h, w), lambda i: (i, 0, 0)),
        out_shape=jax.ShapeDtypeStruct((bc, h, w), jnp.float32),
        scratch_shapes=[pltpu.VMEM((pb, hp, wp), jnp.float32),
                        pltpu.VMEM((pb, hp, wp), jnp.float32)],
    )(alpha, nlev, xf)
    return out.reshape(b, c, h, w)


# speedup vs baseline: 1.2173x; 1.0606x over previous
import functools

import jax
import jax.numpy as jnp
from jax.experimental import pallas as pl
from jax.experimental.pallas import tpu as pltpu

_KS = (1, 3, 5, 7, 9, 11, 13)
_NK = len(_KS)
_PADR = 8
_PADL = 8
_NEG = -jnp.inf


def _means_body(x_ref, o_ref):
    s = jnp.sum(x_ref[...], axis=(1, 2))
    o_ref[0, 0, :] = s


def _router_body(content_ref, w1_ref, b1_ref, w2_ref, b2_ref,
                 alpha_ref, nlev_ref):
    content = content_ref[...]
    hidden = jnp.maximum(
        jax.lax.dot_general(content, w1_ref[...],
                            (((1,), (1,)), ((), ())),
                            preferred_element_type=jnp.float32)
        + b1_ref[0, :][None, :], 0.0)
    glob = jax.lax.dot_general(hidden, w2_ref[...],
                               (((1,), (1,)), ((), ())),
                               preferred_element_type=jnp.float32) \
        + b2_ref[0, :][None, :]
    e = jnp.maximum(glob, 0.0)
    q_s = jnp.clip(jnp.floor(e), 0.0, float(_NK - 2))
    w_big = e - q_s
    w_small = (q_s + 1.0) - e
    for i in range(_NK):
        fi = float(i)
        alpha = jnp.where(q_s == fi, w_small, 0.0) \
            + jnp.where(q_s == fi - 1.0, w_big, 0.0)
        alpha_ref[i, :, :] = alpha
    nlev_ref[...] = (q_s + 1.0).astype(jnp.int32)


def _pool_body(alpha_ref, nlev_ref, x_ref, o_ref, a_ref, b_ref, *, h, w, c,
               cb):
    g = pl.program_id(0)

    hp = h + 2 * _PADR
    wp = w + 2 * _PADL

    a_ref[:, 0:_PADR, :] = jnp.full((cb, _PADR, wp), _NEG, jnp.float32)
    a_ref[:, _PADR + h:, :] = jnp.full((cb, _PADR, wp), _NEG, jnp.float32)
    a_ref[:, :, _PADL - 8:_PADL] = jnp.full((cb, hp, 8), _NEG, jnp.float32)
    a_ref[:, :, _PADL + w:_PADL + w + 8] = jnp.full((cb, hp, 8), _NEG,
                                                    jnp.float32)
    x = x_ref[...]
    a_ref[:, _PADR:_PADR + h, _PADL:_PADL + w] = x

    nlevs = []
    nlev_all = 0
    for j in range(cb):
        bc = g * cb + j
        bi = bc // c
        ci = bc % c
        a0 = alpha_ref[0, bi, ci]
        o_ref[j] = (1.0 + a0) * x[j]
        nl = nlev_ref[bi, ci]
        nlevs.append((nl, bi, ci))
        nlev_all = jnp.maximum(nlev_all, nl)

    for i in range(1, _NK):
        r = (_NK - 1) - i

        @pl.when(i <= nlev_all)
        def _():
            lo = _PADR - r - 1
            hi = _PADR + h + r + 1
            rl = _PADR - r
            rh = _PADR + h + r
            cl = _PADL - r
            ch = _PADL + w + r
            v = a_ref[...]
            b_ref[:, lo:hi, cl:ch] = jnp.maximum(
                jnp.maximum(v[:, lo:hi, cl - 1:ch - 1], v[:, lo:hi, cl:ch]),
                v[:, lo:hi, cl + 1:ch + 1])
            u = b_ref[...]
            a_ref[:, rl:rh, cl:ch] = jnp.maximum(
                jnp.maximum(u[:, lo:rh - 1, cl:ch], u[:, rl:rh, cl:ch]),
                u[:, rl + 1:hi, cl:ch])

            for j in range(cb):
                nl, bi, ci = nlevs[j]

                @pl.when((i >= nl - 1) & (i <= nl))
                def _():
                    ai = alpha_ref[i, bi, ci]
                    o_ref[j] = o_ref[j] + ai * a_ref[j, _PADR:_PADR + h,
                                                     _PADL:_PADL + w]


def kernel(x, W1, b1, W2, b2):
    b, c, h, w = x.shape
    cr = W1.shape[0]
    bc = b * c
    xf = x.reshape(bc, h, w)

    cb = 16
    assert bc % cb == 0
    sums = pl.pallas_call(
        _means_body,
        grid=(bc // cb,),
        in_specs=[pl.BlockSpec((cb, h, w), lambda i: (i, 0, 0))],
        out_specs=pl.BlockSpec((1, 1, cb), lambda i: (i, 0, 0)),
        out_shape=jax.ShapeDtypeStruct((bc // cb, 1, cb), jnp.float32),
    )(xf)
    content = sums.reshape(b, c) * (1.0 / (h * w))

    alpha, nlev = pl.pallas_call(
        _router_body,
        out_shape=(jax.ShapeDtypeStruct((_NK, b, c), jnp.float32),
                   jax.ShapeDtypeStruct((b, c), jnp.int32)),
    )(content, W1, b1.reshape(1, cr), W2, b2.reshape(1, c))

    hp, wp = h + 2 * _PADR, w + 2 * _PADL
    pb = 1
    out = pl.pallas_call(
        functools.partial(_pool_body, h=h, w=w, c=c, cb=pb),
        grid=(bc // pb,),
        in_specs=[
            pl.BlockSpec(memory_space=pltpu.SMEM),
            pl.BlockSpec(memory_space=pltpu.SMEM),
            pl.BlockSpec((pb, h, w), lambda i: (i, 0, 0)),
        ],
        out_specs=pl.BlockSpec((pb, h, w), lambda i: (i, 0, 0)),
        out_shape=jax.ShapeDtypeStruct((bc, h, w), jnp.float32),
        scratch_shapes=[pltpu.VMEM((pb, hp, wp), jnp.float32),
                        pltpu.VMEM((pb, hp, wp), jnp.float32)],
    )(alpha, nlev, xf)
    return out.reshape(b, c, h, w)


# speedup vs baseline: 1.2874x; 1.0576x over previous
import functools

import jax
import jax.numpy as jnp
from jax.experimental import pallas as pl
from jax.experimental.pallas import tpu as pltpu

_KS = (1, 3, 5, 7, 9, 11, 13)
_NK = len(_KS)
_PADR = 8
_PADL = 8
_NEG = -jnp.inf


def _means_body(x_ref, o_ref):
    s = jnp.sum(x_ref[...], axis=(1, 2))
    o_ref[0, 0, :] = s


def _router_body(content_ref, w1_ref, b1_ref, w2_ref, b2_ref,
                 alpha_ref, nlev_ref):
    content = content_ref[...]
    hidden = jnp.maximum(
        jax.lax.dot_general(content, w1_ref[...],
                            (((1,), (1,)), ((), ())),
                            preferred_element_type=jnp.float32)
        + b1_ref[0, :][None, :], 0.0)
    glob = jax.lax.dot_general(hidden, w2_ref[...],
                               (((1,), (1,)), ((), ())),
                               preferred_element_type=jnp.float32) \
        + b2_ref[0, :][None, :]
    e = jnp.maximum(glob, 0.0)
    q_s = jnp.clip(jnp.floor(e), 0.0, float(_NK - 2))
    w_big = e - q_s
    w_small = (q_s + 1.0) - e
    for i in range(_NK):
        fi = float(i)
        alpha = jnp.where(q_s == fi, w_small, 0.0) \
            + jnp.where(q_s == fi - 1.0, w_big, 0.0)
        alpha_ref[i, :, :] = alpha
    nlev_ref[...] = (q_s + 1.0).astype(jnp.int32)


def _pool_body(alpha_ref, nlev_ref, x_ref, o_ref, a_ref, b_ref, *, h, w, c,
               cb):
    g = pl.program_id(0)

    hp = h + 2 * _PADR
    wp = w + 2 * _PADL

    sdt = a_ref.dtype
    a_ref[:, 0:_PADR, :] = jnp.full((cb, _PADR, wp), _NEG, sdt)
    a_ref[:, _PADR + h:, :] = jnp.full((cb, _PADR, wp), _NEG, sdt)
    a_ref[:, :, _PADL - 8:_PADL] = jnp.full((cb, hp, 8), _NEG, sdt)
    a_ref[:, :, _PADL + w:_PADL + w + 8] = jnp.full((cb, hp, 8), _NEG, sdt)
    x = x_ref[...]
    a_ref[:, _PADR:_PADR + h, _PADL:_PADL + w] = x.astype(sdt)

    nlevs = []
    nlev_all = 0
    for j in range(cb):
        bc = g * cb + j
        bi = bc // c
        ci = bc % c
        a0 = alpha_ref[0, bi, ci]
        o_ref[j] = (1.0 + a0) * x[j]
        nl = nlev_ref[bi, ci]
        nlevs.append((nl, bi, ci))
        nlev_all = jnp.maximum(nlev_all, nl)

    for i in range(1, _NK):
        r = (_NK - 1) - i

        @pl.when(i <= nlev_all)
        def _():
            lo = _PADR - r - 1
            hi = _PADR + h + r + 1
            rl = _PADR - r
            rh = _PADR + h + r
            cl = _PADL - r
            ch = _PADL + w + r
            v = a_ref[...]
            b_ref[:, lo:hi, cl:ch] = jnp.maximum(
                jnp.maximum(v[:, lo:hi, cl - 1:ch - 1], v[:, lo:hi, cl:ch]),
                v[:, lo:hi, cl + 1:ch + 1])
            u = b_ref[...]
            a_ref[:, rl:rh, cl:ch] = jnp.maximum(
                jnp.maximum(u[:, lo:rh - 1, cl:ch], u[:, rl:rh, cl:ch]),
                u[:, rl + 1:hi, cl:ch])

            for j in range(cb):
                nl, bi, ci = nlevs[j]

                @pl.when((i >= nl - 1) & (i <= nl))
                def _():
                    ai = alpha_ref[i, bi, ci]
                    p = a_ref[j, _PADR:_PADR + h,
                              _PADL:_PADL + w].astype(jnp.float32)
                    o_ref[j] = o_ref[j] + ai * p


def kernel(x, W1, b1, W2, b2):
    b, c, h, w = x.shape
    cr = W1.shape[0]
    bc = b * c
    xf = x.reshape(bc, h, w)

    cb = 16
    assert bc % cb == 0
    sums = pl.pallas_call(
        _means_body,
        grid=(bc // cb,),
        in_specs=[pl.BlockSpec((cb, h, w), lambda i: (i, 0, 0))],
        out_specs=pl.BlockSpec((1, 1, cb), lambda i: (i, 0, 0)),
        out_shape=jax.ShapeDtypeStruct((bc // cb, 1, cb), jnp.float32),
    )(xf)
    content = sums.reshape(b, c) * (1.0 / (h * w))

    alpha, nlev = pl.pallas_call(
        _router_body,
        out_shape=(jax.ShapeDtypeStruct((_NK, b, c), jnp.float32),
                   jax.ShapeDtypeStruct((b, c), jnp.int32)),
    )(content, W1, b1.reshape(1, cr), W2, b2.reshape(1, c))

    hp, wp = h + 2 * _PADR, w + 2 * _PADL
    pb = 1
    out = pl.pallas_call(
        functools.partial(_pool_body, h=h, w=w, c=c, cb=pb),
        grid=(bc // pb,),
        in_specs=[
            pl.BlockSpec(memory_space=pltpu.SMEM),
            pl.BlockSpec(memory_space=pltpu.SMEM),
            pl.BlockSpec((pb, h, w), lambda i: (i, 0, 0)),
        ],
        out_specs=pl.BlockSpec((pb, h, w), lambda i: (i, 0, 0)),
        out_shape=jax.ShapeDtypeStruct((bc, h, w), jnp.float32),
        scratch_shapes=[pltpu.VMEM((pb, hp, wp), jnp.bfloat16),
                        pltpu.VMEM((pb, hp, wp), jnp.bfloat16)],
    )(alpha, nlev, xf)
    return out.reshape(b, c, h, w)


# speedup vs baseline: 1.4594x; 1.1336x over previous
import functools

import jax
import jax.numpy as jnp
from jax.experimental import pallas as pl
from jax.experimental.pallas import tpu as pltpu

_KS = (1, 3, 5, 7, 9, 11, 13)
_NK = len(_KS)
_PADR = 8
_PADL = 8
_NEG = -jnp.inf


def _means_body(x_ref, o_ref):
    s = jnp.sum(x_ref[...], axis=(1, 2))
    o_ref[0, 0, :] = s


def _router_body(content_ref, w1_ref, b1_ref, w2_ref, b2_ref,
                 alpha_ref, nlev_ref):
    content = content_ref[...]
    hidden = jnp.maximum(
        jax.lax.dot_general(content, w1_ref[...],
                            (((1,), (1,)), ((), ())),
                            preferred_element_type=jnp.float32)
        + b1_ref[0, :][None, :], 0.0)
    glob = jax.lax.dot_general(hidden, w2_ref[...],
                               (((1,), (1,)), ((), ())),
                               preferred_element_type=jnp.float32) \
        + b2_ref[0, :][None, :]
    e = jnp.maximum(glob, 0.0)
    q_s = jnp.clip(jnp.floor(e), 0.0, float(_NK - 2))
    w_big = e - q_s
    w_small = (q_s + 1.0) - e
    for i in range(_NK):
        fi = float(i)
        alpha = jnp.where(q_s == fi, w_small, 0.0) \
            + jnp.where(q_s == fi - 1.0, w_big, 0.0)
        alpha_ref[i, :, :] = alpha
    nlev_ref[...] = (q_s + 1.0).astype(jnp.int32)


def _pool_body(alpha_ref, nlev_ref, x_ref, o_ref, a_ref, b_ref, *, h, w, c,
               cb):
    g = pl.program_id(0)

    hp = h + 2 * _PADR
    wp = w + 2 * _PADL

    sdt = a_ref.dtype
    a_ref[:, 0:_PADR, :] = jnp.full((cb, _PADR, wp), _NEG, sdt)
    a_ref[:, _PADR + h:, :] = jnp.full((cb, _PADR, wp), _NEG, sdt)
    a_ref[:, :, _PADL - 8:_PADL] = jnp.full((cb, hp, 8), _NEG, sdt)
    a_ref[:, :, _PADL + w:_PADL + w + 8] = jnp.full((cb, hp, 8), _NEG, sdt)
    x = x_ref[...]
    a_ref[:, _PADR:_PADR + h, _PADL:_PADL + w] = x.astype(sdt)

    nlevs = []
    nlev_all = 0
    for j in range(cb):
        bc = g * cb + j
        bi = bc // c
        ci = bc % c
        a0 = alpha_ref[0, bi, ci]
        o_ref[j] = (1.0 + a0) * x[j]
        nl = nlev_ref[bi, ci]
        nlevs.append((nl, bi, ci))
        nlev_all = jnp.maximum(nlev_all, nl)

    for i in range(1, _NK):
        r = (_NK - 1) - i

        @pl.when(i <= nlev_all)
        def _():
            lo = _PADR - r - 1
            hi = _PADR + h + r + 1
            rl = _PADR - r
            rh = _PADR + h + r
            cl = _PADL - r
            ch = _PADL + w + r
            v = a_ref[...]
            b_ref[:, lo:hi, cl:ch] = jnp.maximum(
                jnp.maximum(v[:, lo:hi, cl - 1:ch - 1], v[:, lo:hi, cl:ch]),
                v[:, lo:hi, cl + 1:ch + 1])
            u = b_ref[...]
            a_ref[:, rl:rh, cl:ch] = jnp.maximum(
                jnp.maximum(u[:, lo:rh - 1, cl:ch], u[:, rl:rh, cl:ch]),
                u[:, rl + 1:hi, cl:ch])

            for j in range(cb):
                nl, bi, ci = nlevs[j]

                @pl.when((i >= nl - 1) & (i <= nl))
                def _():
                    ai = alpha_ref[i, bi, ci]
                    p = a_ref[j, _PADR:_PADR + h,
                              _PADL:_PADL + w].astype(jnp.float32)
                    o_ref[j] = o_ref[j] + ai * p


def kernel(x, W1, b1, W2, b2):
    b, c, h, w = x.shape
    cr = W1.shape[0]
    bc = b * c
    xf = x.reshape(bc, h, w)

    cb = 16
    assert bc % cb == 0
    sums = pl.pallas_call(
        _means_body,
        grid=(bc // cb,),
        in_specs=[pl.BlockSpec((cb, h, w), lambda i: (i, 0, 0))],
        out_specs=pl.BlockSpec((1, 1, cb), lambda i: (i, 0, 0)),
        out_shape=jax.ShapeDtypeStruct((bc // cb, 1, cb), jnp.float32),
    )(xf)
    content = sums.reshape(b, c) * (1.0 / (h * w))

    alpha, nlev = pl.pallas_call(
        _router_body,
        out_shape=(jax.ShapeDtypeStruct((_NK, b, c), jnp.float32),
                   jax.ShapeDtypeStruct((b, c), jnp.int32)),
    )(content, W1, b1.reshape(1, cr), W2, b2.reshape(1, c))

    hp, wp = h + 2 * _PADR, w + 2 * _PADL
    pb = 2
    out = pl.pallas_call(
        functools.partial(_pool_body, h=h, w=w, c=c, cb=pb),
        grid=(bc // pb,),
        in_specs=[
            pl.BlockSpec(memory_space=pltpu.SMEM),
            pl.BlockSpec(memory_space=pltpu.SMEM),
            pl.BlockSpec((pb, h, w), lambda i: (i, 0, 0)),
        ],
        out_specs=pl.BlockSpec((pb, h, w), lambda i: (i, 0, 0)),
        out_shape=jax.ShapeDtypeStruct((bc, h, w), jnp.float32),
        scratch_shapes=[pltpu.VMEM((pb, hp, wp), jnp.bfloat16),
                        pltpu.VMEM((pb, hp, wp), jnp.bfloat16)],
    )(alpha, nlev, xf)
    return out.reshape(b, c, h, w)
